# baseline (speedup 1.0000x reference)
import functools

import jax
import jax.numpy as jnp
from jax import lax
from jax.experimental import pallas as pl
from jax.experimental.pallas import tpu as pltpu
from jax.experimental.pallas import tpu_sc as plsc

NUM_BUCKETS = 2048
HALF_DIM = 32
EMBEDDING_DIM = 64

_info = plsc.get_sparse_core_info()
_NC, _NS, _L = _info.num_cores, _info.num_subcores, _info.num_lanes
_NW = _NC * _NS

_STAGE_BATCHES = 8
_IDX_SPLITS = ((0, 104), (104, 96))


def _proj_body(t_ref, w_ref, b_ref, o_ref):
    o_ref[...] = (
        jnp.dot(t_ref[...], w_ref[...], preferred_element_type=jnp.float32)
        + b_ref[...]
    )


def _project_table(bucket_table, W, b):
    return pl.pallas_call(
        _proj_body,
        out_shape=jax.ShapeDtypeStruct((NUM_BUCKETS, EMBEDDING_DIM), jnp.float32),
    )(bucket_table, W, b.reshape(1, EMBEDDING_DIM))


def _make_gather(B, L):
    assert B % (_NW * _STAGE_BATCHES) == 0 and L == sum(n for _, n in _IDX_SPLITS)
    batches_per_worker = B // _NW
    stages_per_worker = batches_per_worker // _STAGE_BATCHES
    mesh = plsc.VectorSubcoreMesh(core_axis_name="c", subcore_axis_name="s")

    @functools.partial(
        pl.kernel,
        out_type=jax.ShapeDtypeStruct((B, L, EMBEDDING_DIM), jnp.float32),
        mesh=mesh,
        scratch_types=[
            pltpu.VMEM((_STAGE_BATCHES, L), jnp.int32),
            pltpu.VMEM((L, EMBEDDING_DIM), jnp.float32),
            pltpu.SemaphoreType.DMA,
        ],
        compiler_params=pltpu.CompilerParams(use_tc_tiling_on_sc=False),
    )
    def gather_kernel(p_hbm, x_hbm, out_hbm, idx_v, rows_v, sem):
        wid = lax.axis_index("s") * _NC + lax.axis_index("c")
        batch0 = wid * batches_per_worker

        def stage_body(st, carry):
            bb = batch0 + st * _STAGE_BATCHES
            pltpu.sync_copy(x_hbm.at[pl.ds(bb, _STAGE_BATCHES)], idx_v)
            for r in range(_STAGE_BATCHES):
                for g0 in list(range(0, L - _L + 1, _L)) + [L - _L]:
                    sl = pl.ds(g0, _L)
                    idx_v[r, sl] = lax.bitwise_and(idx_v[r, sl], NUM_BUCKETS - 1)
            for r in range(_STAGE_BATCHES):
                copies = [
                    pltpu.async_copy(
                        p_hbm.at[idx_v.at[r, pl.ds(off, n)]],
                        rows_v.at[pl.ds(off, n)],
                        sem,
                    )
                    for off, n in _IDX_SPLITS
                ]
                for cp in copies:
                    cp.wait()
                pltpu.sync_copy(rows_v, out_hbm.at[bb + r])
            return carry

        lax.fori_loop(0, stages_per_worker, stage_body, 0)

    return gather_kernel


def kernel(x, bucket_table, W, b):
    B, L = x.shape
    P = _project_table(bucket_table, W, b)
    return _make_gather(B, L)(P, x.astype(jnp.int32))
